# baseline (speedup 1.0000x reference)
import functools

import jax
import jax.numpy as jnp
from jax import lax
from jax.experimental import pallas as pl
from jax.experimental.pallas import tpu as pltpu
from jax.experimental.pallas import tpu_sc as plsc

_N = 10000
_E = 160000
_H = 256
_G = 128
_MID = 32
_OUT = 64
_HALF = 128
_NSUB = 16
_EROWS_PAD = 1280
_CPS = _EROWS_PAD // _NSUB
_ROWS_PER_SUB = 624
_TAIL = _N - _NSUB * _ROWS_PER_SUB
_TBL = 2 * _N + 8
_EPS = 1e-5


def _sc_segsum_body(h_hbm, idx_hbm, out_hbm, acc_sh, idx_v, rows_v):
    c = lax.axis_index("c")
    s = lax.axis_index("s")

    pltpu.sync_copy(h_hbm.at[pl.ds(c * _N + s * _ROWS_PER_SUB, _ROWS_PER_SUB)],
                    acc_sh.at[pl.ds(s * _ROWS_PER_SUB, _ROWS_PER_SUB)])

    @pl.when(s == _NSUB - 1)
    def _():
        pltpu.sync_copy(h_hbm.at[pl.ds(c * _N + _NSUB * _ROWS_PER_SUB, _TAIL)],
                        acc_sh.at[pl.ds(_NSUB * _ROWS_PER_SUB, _TAIL)])

    plsc.subcore_barrier()

    base = c * _EROWS_PAD + s * _CPS

    @pl.loop(0, _CPS)
    def _edge_chunk(k):
        pltpu.sync_copy(idx_hbm.at[base + k], idx_v)
        pltpu.sync_copy(h_hbm.at[idx_v.at[0]], rows_v)
        pltpu.sync_copy(rows_v, acc_sh.at[idx_v.at[1]], add=True)

    plsc.subcore_barrier()
    pltpu.sync_copy(acc_sh.at[pl.ds(s * _ROWS_PER_SUB, _ROWS_PER_SUB)],
                    out_hbm.at[pl.ds(c * _N + s * _ROWS_PER_SUB, _ROWS_PER_SUB)])

    @pl.when(s == _NSUB - 1)
    def _():
        pltpu.sync_copy(acc_sh.at[pl.ds(_NSUB * _ROWS_PER_SUB, _TAIL)],
                        out_hbm.at[pl.ds(c * _N + _NSUB * _ROWS_PER_SUB, _TAIL)])


@functools.cache
def _get_sc_segsum():
    return functools.partial(
        pl.kernel,
        out_type=jax.ShapeDtypeStruct((_TBL, _HALF), jnp.float32),
        mesh=plsc.VectorSubcoreMesh(core_axis_name="c", subcore_axis_name="s"),
        scratch_types=[
            pltpu.VMEM_SHARED((_N, _HALF), jnp.float32),
            pltpu.VMEM((2, 128), jnp.int32),
            pltpu.VMEM((128, _HALF), jnp.float32),
        ],
    )(_sc_segsum_body)


def _mlp_stage_body(x_ref, w_ref, b_ref, g_ref, be_ref, o_ref):
    xl = x_ref[:_N]
    xr = x_ref[_N:2 * _N]
    for j in range(2):
        sl = slice(j * _HALF, (j + 1) * _HALF)
        y = (jnp.dot(xl, w_ref[:_HALF, sl], preferred_element_type=jnp.float32)
             + jnp.dot(xr, w_ref[_HALF:, sl], preferred_element_type=jnp.float32)
             + b_ref[:, sl])
        m = jnp.mean(y, axis=0, keepdims=True)
        v = jnp.mean((y - m) ** 2, axis=0, keepdims=True)
        hn = (y - m) / jnp.sqrt(v + _EPS) * g_ref[:, sl] + be_ref[:, sl]
        o_ref[j * _N:(j + 1) * _N] = jnp.maximum(hn, 0.0)
    o_ref[2 * _N:] = jnp.zeros((_TBL - 2 * _N, _HALF), jnp.float32)


_mlp_stage = pl.pallas_call(
    _mlp_stage_body,
    out_shape=jax.ShapeDtypeStruct((_TBL, _HALF), jnp.float32),
)


def _pool_body(xst_ref, hst_ref, batch_ref,
               wp0_ref, bp0_ref, wp3_ref, bp3_ref, wo_ref, bo_ref, o_ref):
    gi = lax.broadcasted_iota(jnp.int32, (1, _G), 1)
    m = (batch_ref[...] == gi).astype(jnp.float32)
    dn = (((0,), (0,)), ((), ()))

    def pool_proj(st_ref, w_ref):
        pleft = lax.dot_general(m, st_ref[:_N], dn,
                                preferred_element_type=jnp.float32)
        pright = lax.dot_general(m, st_ref[_N:2 * _N], dn,
                                 preferred_element_type=jnp.float32)
        return (jnp.dot(pleft, w_ref[:_HALF], preferred_element_type=jnp.float32)
                + jnp.dot(pright, w_ref[_HALF:], preferred_element_type=jnp.float32))

    oh = (pool_proj(xst_ref, wp0_ref) + pool_proj(hst_ref, wp3_ref)
          + bp0_ref[...] + bp3_ref[...])
    oh = jnp.maximum(oh, 0.0)
    o_ref[...] = jnp.dot(oh, wo_ref[...],
                         preferred_element_type=jnp.float32) + bo_ref[...]


_pool = pl.pallas_call(
    _pool_body,
    out_shape=jax.ShapeDtypeStruct((_G, _OUT), jnp.float32),
)


def kernel(x, edge_index, batch, atten_edge_index,
           l0_W1, l0_b1, l0_g1, l0_be1, l0_W2, l0_b2, l0_g2, l0_be2,
           l1_W1, l1_b1, l1_g1, l1_be1, l1_W2, l1_b2, l1_g2, l1_be2,
           l2_W1, l2_b1, l2_g1, l2_be1, l2_W2, l2_b2, l2_g2, l2_be2,
           Wp0, bp0, Wp3, bp3, Wo, bo):
    del atten_edge_index

    pad = _EROWS_PAD * 128 - _E
    fill = jnp.full((pad,), 2 * _N, dtype=jnp.int32)
    col = edge_index[1]
    row_pad = jnp.concatenate(
        [edge_index[0], jnp.zeros((pad,), dtype=jnp.int32)]
    ).reshape(_EROWS_PAD, 1, 128)
    idx_cat = jnp.concatenate([
        jnp.concatenate(
            [jnp.concatenate([col, fill]).reshape(_EROWS_PAD, 1, 128), row_pad],
            axis=1),
        jnp.concatenate(
            [jnp.concatenate([col + _N, fill]).reshape(_EROWS_PAD, 1, 128),
             row_pad],
            axis=1),
    ])

    x_st = jnp.concatenate(
        [x[:, :_HALF], x[:, _HALF:], jnp.zeros((_TBL - 2 * _N, _HALF), x.dtype)],
        axis=0)
    batch2 = batch.reshape(_N, 1)

    layers = [
        (l0_W1, l0_b1, l0_g1, l0_be1, l0_W2, l0_b2, l0_g2, l0_be2),
        (l1_W1, l1_b1, l1_g1, l1_be1, l1_W2, l1_b2, l1_g2, l1_be2),
        (l2_W1, l2_b1, l2_g1, l2_be1, l2_W2, l2_b2, l2_g2, l2_be2),
    ]

    def r1(v):
        return v.reshape(1, -1)

    sc_segsum = _get_sc_segsum()
    h_st = x_st
    for (W1, b1, g1, be1, W2, b2, g2, be2) in layers:
        a_st = sc_segsum(h_st, idx_cat)
        t_st = _mlp_stage(a_st, W1, r1(b1), r1(g1), r1(be1))
        h_st = _mlp_stage(t_st, W2, r1(b2), r1(g2), r1(be2))

    return _pool(x_st, h_st, batch2,
                 Wp0, r1(bp0), Wp3, r1(bp3), Wo, r1(bo))


# speedup vs baseline: 1.7246x; 1.7246x over previous
import functools

import jax
import jax.numpy as jnp
from jax import lax
from jax.experimental import pallas as pl
from jax.experimental.pallas import tpu as pltpu
from jax.experimental.pallas import tpu_sc as plsc

_N = 10000
_E = 160000
_H = 256
_G = 128
_MID = 32
_OUT = 64
_HALF = 128
_NSUB = 16
_EROWS = _E // 128
_ROWS_PER_SUB = 624
_TAIL = _N - _NSUB * _ROWS_PER_SUB
_CHUNKS_PER_SUB = -(-_EROWS // _NSUB)
_EPS = 1e-5


def _sc_segsum_body(h_hbm, col_hbm, row_hbm, out_hbm,
                    acc_sh, colidx_v, rowidx_v, rows_v):
    c = lax.axis_index("c")
    s = lax.axis_index("s")

    pltpu.sync_copy(h_hbm.at[pl.ds(c * _N + s * _ROWS_PER_SUB, _ROWS_PER_SUB)],
                    acc_sh.at[pl.ds(s * _ROWS_PER_SUB, _ROWS_PER_SUB)])

    @pl.when(s == _NSUB - 1)
    def _():
        pltpu.sync_copy(h_hbm.at[pl.ds(c * _N + _NSUB * _ROWS_PER_SUB, _TAIL)],
                        acc_sh.at[pl.ds(_NSUB * _ROWS_PER_SUB, _TAIL)])

    plsc.subcore_barrier()

    @pl.loop(0, _CHUNKS_PER_SUB)
    def _edge_chunk(k):
        r = s + _NSUB * k

        @pl.when(r < _EROWS)
        def _():
            pltpu.sync_copy(col_hbm.at[r], colidx_v)
            pltpu.sync_copy(row_hbm.at[r], rowidx_v)

            @pl.when(c == 1)
            def _():
                @pl.loop(0, 128, step=16)
                def _(j):
                    colidx_v[0, pl.ds(j, 16)] = colidx_v[0, pl.ds(j, 16)] + _N

            pltpu.sync_copy(h_hbm.at[colidx_v.at[0]], rows_v)
            pltpu.sync_copy(rows_v, acc_sh.at[rowidx_v.at[0]], add=True)

    plsc.subcore_barrier()
    pltpu.sync_copy(acc_sh.at[pl.ds(s * _ROWS_PER_SUB, _ROWS_PER_SUB)],
                    out_hbm.at[pl.ds(c * _N + s * _ROWS_PER_SUB, _ROWS_PER_SUB)])

    @pl.when(s == _NSUB - 1)
    def _():
        pltpu.sync_copy(acc_sh.at[pl.ds(_NSUB * _ROWS_PER_SUB, _TAIL)],
                        out_hbm.at[pl.ds(c * _N + _NSUB * _ROWS_PER_SUB, _TAIL)])


@functools.cache
def _get_sc_segsum():
    return functools.partial(
        pl.kernel,
        out_type=jax.ShapeDtypeStruct((2 * _N, _HALF), jnp.float32),
        mesh=plsc.VectorSubcoreMesh(core_axis_name="c", subcore_axis_name="s"),
        scratch_types=[
            pltpu.VMEM_SHARED((_N, _HALF), jnp.float32),
            pltpu.VMEM((1, 128), jnp.int32),
            pltpu.VMEM((1, 128), jnp.int32),
            pltpu.VMEM((128, _HALF), jnp.float32),
        ],
    )(_sc_segsum_body)


def _mlp_stage_body(x_ref, w_ref, b_ref, g_ref, be_ref, o_ref):
    xl = x_ref[:_N]
    xr = x_ref[_N:]
    for j in range(2):
        sl = slice(j * _HALF, (j + 1) * _HALF)
        y = (jnp.dot(xl, w_ref[:_HALF, sl], preferred_element_type=jnp.float32)
             + jnp.dot(xr, w_ref[_HALF:, sl], preferred_element_type=jnp.float32)
             + b_ref[:, sl])
        m = jnp.mean(y, axis=0, keepdims=True)
        v = jnp.mean((y - m) ** 2, axis=0, keepdims=True)
        hn = (y - m) / jnp.sqrt(v + _EPS) * g_ref[:, sl] + be_ref[:, sl]
        o_ref[j * _N:(j + 1) * _N] = jnp.maximum(hn, 0.0)


_mlp_stage = pl.pallas_call(
    _mlp_stage_body,
    out_shape=jax.ShapeDtypeStruct((2 * _N, _HALF), jnp.float32),
)


def _pool_body(xst_ref, hst_ref, batch_ref,
               wp0_ref, bp0_ref, wp3_ref, bp3_ref, wo_ref, bo_ref, o_ref):
    gi = lax.broadcasted_iota(jnp.int32, (1, _G), 1)
    m = (batch_ref[...] == gi).astype(jnp.float32)
    dn = (((0,), (0,)), ((), ()))

    def pool_proj(st_ref, w_ref):
        pleft = lax.dot_general(m, st_ref[:_N], dn,
                                preferred_element_type=jnp.float32)
        pright = lax.dot_general(m, st_ref[_N:], dn,
                                 preferred_element_type=jnp.float32)
        return (jnp.dot(pleft, w_ref[:_HALF], preferred_element_type=jnp.float32)
                + jnp.dot(pright, w_ref[_HALF:], preferred_element_type=jnp.float32))

    oh = (pool_proj(xst_ref, wp0_ref) + pool_proj(hst_ref, wp3_ref)
          + bp0_ref[...] + bp3_ref[...])
    oh = jnp.maximum(oh, 0.0)
    o_ref[...] = jnp.dot(oh, wo_ref[...],
                         preferred_element_type=jnp.float32) + bo_ref[...]


_pool = pl.pallas_call(
    _pool_body,
    out_shape=jax.ShapeDtypeStruct((_G, _OUT), jnp.float32),
)


def kernel(x, edge_index, batch, atten_edge_index,
           l0_W1, l0_b1, l0_g1, l0_be1, l0_W2, l0_b2, l0_g2, l0_be2,
           l1_W1, l1_b1, l1_g1, l1_be1, l1_W2, l1_b2, l1_g2, l1_be2,
           l2_W1, l2_b1, l2_g1, l2_be1, l2_W2, l2_b2, l2_g2, l2_be2,
           Wp0, bp0, Wp3, bp3, Wo, bo):
    del atten_edge_index
    row = edge_index[0].reshape(_EROWS, 1, 128)
    col = edge_index[1].reshape(_EROWS, 1, 128)
    x_st = jnp.concatenate([x[:, :_HALF], x[:, _HALF:]], axis=0)
    batch2 = batch.reshape(_N, 1)

    layers = [
        (l0_W1, l0_b1, l0_g1, l0_be1, l0_W2, l0_b2, l0_g2, l0_be2),
        (l1_W1, l1_b1, l1_g1, l1_be1, l1_W2, l1_b2, l1_g2, l1_be2),
        (l2_W1, l2_b1, l2_g1, l2_be1, l2_W2, l2_b2, l2_g2, l2_be2),
    ]

    def r1(v):
        return v.reshape(1, -1)

    sc_segsum = _get_sc_segsum()
    h_st = x_st
    for (W1, b1, g1, be1, W2, b2, g2, be2) in layers:
        a_st = sc_segsum(h_st, col, row)
        t_st = _mlp_stage(a_st, W1, r1(b1), r1(g1), r1(be1))
        h_st = _mlp_stage(t_st, W2, r1(b2), r1(g2), r1(be2))

    return _pool(x_st, h_st, batch2,
                 Wp0, r1(bp0), Wp3, r1(bp3), Wo, r1(bo))


# speedup vs baseline: 2.1450x; 1.2438x over previous
import functools

import jax
import jax.numpy as jnp
from jax import lax
from jax.experimental import pallas as pl
from jax.experimental.pallas import tpu as pltpu
from jax.experimental.pallas import tpu_sc as plsc

_N = 10000
_E = 160000
_H = 256
_G = 128
_MID = 32
_OUT = 64
_HALF = 128
_NSUB = 16
_ECHUNKS = _E // 256
_ROWS_PER_SUB = 624
_TAIL = _N - _NSUB * _ROWS_PER_SUB
_CHUNKS_PER_SUB = -(-_ECHUNKS // _NSUB)
_EPS = 1e-5


def _sc_segsum_body(h_hbm, col_hbm, row_hbm, out_hbm,
                    acc_sh, colidx_v, rowidx_v, rows_v):
    c = lax.axis_index("c")
    s = lax.axis_index("s")

    pltpu.sync_copy(h_hbm.at[pl.ds(c * _N + s * _ROWS_PER_SUB, _ROWS_PER_SUB)],
                    acc_sh.at[pl.ds(s * _ROWS_PER_SUB, _ROWS_PER_SUB)])

    @pl.when(s == _NSUB - 1)
    def _():
        pltpu.sync_copy(h_hbm.at[pl.ds(c * _N + _NSUB * _ROWS_PER_SUB, _TAIL)],
                        acc_sh.at[pl.ds(_NSUB * _ROWS_PER_SUB, _TAIL)])

    plsc.subcore_barrier()

    @pl.loop(0, _CHUNKS_PER_SUB)
    def _edge_chunk(k):
        r = s + _NSUB * k

        @pl.when(r < _ECHUNKS)
        def _():
            pltpu.sync_copy(col_hbm.at[pl.ds(r * 256, 256)], colidx_v)
            pltpu.sync_copy(row_hbm.at[pl.ds(r * 256, 256)], rowidx_v)

            @pl.when(c == 1)
            def _():
                @pl.loop(0, 256, step=16)
                def _(j):
                    colidx_v[pl.ds(j, 16)] = colidx_v[pl.ds(j, 16)] + _N

            pltpu.sync_copy(h_hbm.at[colidx_v], rows_v)
            pltpu.sync_copy(rows_v, acc_sh.at[rowidx_v], add=True)

    plsc.subcore_barrier()
    pltpu.sync_copy(acc_sh.at[pl.ds(s * _ROWS_PER_SUB, _ROWS_PER_SUB)],
                    out_hbm.at[pl.ds(c * _N + s * _ROWS_PER_SUB, _ROWS_PER_SUB)])

    @pl.when(s == _NSUB - 1)
    def _():
        pltpu.sync_copy(acc_sh.at[pl.ds(_NSUB * _ROWS_PER_SUB, _TAIL)],
                        out_hbm.at[pl.ds(c * _N + _NSUB * _ROWS_PER_SUB, _TAIL)])


@functools.cache
def _get_sc_segsum():
    return functools.partial(
        pl.kernel,
        out_type=jax.ShapeDtypeStruct((2 * _N, _HALF), jnp.float32),
        mesh=plsc.VectorSubcoreMesh(core_axis_name="c", subcore_axis_name="s"),
        scratch_types=[
            pltpu.VMEM_SHARED((_N, _HALF), jnp.float32),
            pltpu.VMEM((256,), jnp.int32),
            pltpu.VMEM((256,), jnp.int32),
            pltpu.VMEM((256, _HALF), jnp.float32),
        ],
    )(_sc_segsum_body)


def _mlp_stage_body(x_ref, w_ref, b_ref, g_ref, be_ref, o_ref):
    xl = x_ref[:_N]
    xr = x_ref[_N:]
    for j in range(2):
        sl = slice(j * _HALF, (j + 1) * _HALF)
        y = (jnp.dot(xl, w_ref[:_HALF, sl], preferred_element_type=jnp.float32)
             + jnp.dot(xr, w_ref[_HALF:, sl], preferred_element_type=jnp.float32)
             + b_ref[:, sl])
        m = jnp.mean(y, axis=0, keepdims=True)
        v = jnp.mean((y - m) ** 2, axis=0, keepdims=True)
        hn = (y - m) / jnp.sqrt(v + _EPS) * g_ref[:, sl] + be_ref[:, sl]
        o_ref[j * _N:(j + 1) * _N] = jnp.maximum(hn, 0.0)


_mlp_stage = pl.pallas_call(
    _mlp_stage_body,
    out_shape=jax.ShapeDtypeStruct((2 * _N, _HALF), jnp.float32),
)


def _pool_body(xst_ref, hst_ref, batch_ref,
               wp0_ref, bp0_ref, wp3_ref, bp3_ref, wo_ref, bo_ref, o_ref):
    gi = lax.broadcasted_iota(jnp.int32, (1, _G), 1)
    m = (batch_ref[...] == gi).astype(jnp.float32)
    dn = (((0,), (0,)), ((), ()))

    def pool_proj(st_ref, w_ref):
        pleft = lax.dot_general(m, st_ref[:_N], dn,
                                preferred_element_type=jnp.float32)
        pright = lax.dot_general(m, st_ref[_N:], dn,
                                 preferred_element_type=jnp.float32)
        return (jnp.dot(pleft, w_ref[:_HALF], preferred_element_type=jnp.float32)
                + jnp.dot(pright, w_ref[_HALF:], preferred_element_type=jnp.float32))

    oh = (pool_proj(xst_ref, wp0_ref) + pool_proj(hst_ref, wp3_ref)
          + bp0_ref[...] + bp3_ref[...])
    oh = jnp.maximum(oh, 0.0)
    o_ref[...] = jnp.dot(oh, wo_ref[...],
                         preferred_element_type=jnp.float32) + bo_ref[...]


_pool = pl.pallas_call(
    _pool_body,
    out_shape=jax.ShapeDtypeStruct((_G, _OUT), jnp.float32),
)


def kernel(x, edge_index, batch, atten_edge_index,
           l0_W1, l0_b1, l0_g1, l0_be1, l0_W2, l0_b2, l0_g2, l0_be2,
           l1_W1, l1_b1, l1_g1, l1_be1, l1_W2, l1_b2, l1_g2, l1_be2,
           l2_W1, l2_b1, l2_g1, l2_be1, l2_W2, l2_b2, l2_g2, l2_be2,
           Wp0, bp0, Wp3, bp3, Wo, bo):
    del atten_edge_index
    row = edge_index[0]
    col = edge_index[1]
    x_st = jnp.concatenate([x[:, :_HALF], x[:, _HALF:]], axis=0)
    batch2 = batch.reshape(_N, 1)

    layers = [
        (l0_W1, l0_b1, l0_g1, l0_be1, l0_W2, l0_b2, l0_g2, l0_be2),
        (l1_W1, l1_b1, l1_g1, l1_be1, l1_W2, l1_b2, l1_g2, l1_be2),
        (l2_W1, l2_b1, l2_g1, l2_be1, l2_W2, l2_b2, l2_g2, l2_be2),
    ]

    def r1(v):
        return v.reshape(1, -1)

    sc_segsum = _get_sc_segsum()
    h_st = x_st
    for (W1, b1, g1, be1, W2, b2, g2, be2) in layers:
        a_st = sc_segsum(h_st, col, row)
        t_st = _mlp_stage(a_st, W1, r1(b1), r1(g1), r1(be1))
        h_st = _mlp_stage(t_st, W2, r1(b2), r1(g2), r1(be2))

    return _pool(x_st, h_st, batch2,
                 Wp0, r1(bp0), Wp3, r1(bp3), Wo, r1(bo))


# speedup vs baseline: 2.1520x; 1.0032x over previous
import functools

import jax
import jax.numpy as jnp
from jax import lax
from jax.experimental import pallas as pl
from jax.experimental.pallas import tpu as pltpu
from jax.experimental.pallas import tpu_sc as plsc

_N = 10000
_E = 160000
_H = 256
_G = 128
_MID = 32
_OUT = 64
_HALF = 128
_NSUB = 16
_ECHUNKS = _E // 256
_ROWS_PER_SUB = 624
_TAIL = _N - _NSUB * _ROWS_PER_SUB
_CHUNKS_PER_SUB = -(-_ECHUNKS // _NSUB)
_EPS = 1e-5


def _sc_segsum_body(h_hbm, col_hbm, row_hbm, out_hbm,
                    acc_sh, colidx_v, rowidx_v, rows_v):
    c = lax.axis_index("c")
    s = lax.axis_index("s")

    pltpu.sync_copy(h_hbm.at[pl.ds(c * _N + s * _ROWS_PER_SUB, _ROWS_PER_SUB)],
                    acc_sh.at[pl.ds(s * _ROWS_PER_SUB, _ROWS_PER_SUB)])

    @pl.when(s == _NSUB - 1)
    def _():
        pltpu.sync_copy(h_hbm.at[pl.ds(c * _N + _NSUB * _ROWS_PER_SUB, _TAIL)],
                        acc_sh.at[pl.ds(_NSUB * _ROWS_PER_SUB, _TAIL)])

    plsc.subcore_barrier()

    @pl.loop(0, _CHUNKS_PER_SUB)
    def _edge_chunk(k):
        r = s + _NSUB * k

        @pl.when(r < _ECHUNKS)
        def _():
            pltpu.sync_copy(col_hbm.at[pl.ds(c * _E + r * 256, 256)], colidx_v)
            pltpu.sync_copy(row_hbm.at[pl.ds(r * 256, 256)], rowidx_v)
            pltpu.sync_copy(h_hbm.at[colidx_v], rows_v)
            pltpu.sync_copy(rows_v, acc_sh.at[rowidx_v], add=True)

    plsc.subcore_barrier()
    pltpu.sync_copy(acc_sh.at[pl.ds(s * _ROWS_PER_SUB, _ROWS_PER_SUB)],
                    out_hbm.at[pl.ds(c * _N + s * _ROWS_PER_SUB, _ROWS_PER_SUB)])

    @pl.when(s == _NSUB - 1)
    def _():
        pltpu.sync_copy(acc_sh.at[pl.ds(_NSUB * _ROWS_PER_SUB, _TAIL)],
                        out_hbm.at[pl.ds(c * _N + _NSUB * _ROWS_PER_SUB, _TAIL)])


@functools.cache
def _get_sc_segsum():
    return functools.partial(
        pl.kernel,
        out_type=jax.ShapeDtypeStruct((2 * _N, _HALF), jnp.float32),
        mesh=plsc.VectorSubcoreMesh(core_axis_name="c", subcore_axis_name="s"),
        scratch_types=[
            pltpu.VMEM_SHARED((_N, _HALF), jnp.float32),
            pltpu.VMEM((256,), jnp.int32),
            pltpu.VMEM((256,), jnp.int32),
            pltpu.VMEM((256, _HALF), jnp.float32),
        ],
    )(_sc_segsum_body)


def _mlp_stage_body(x_ref, w_ref, b_ref, g_ref, be_ref, o_ref):
    xl = x_ref[:_N]
    xr = x_ref[_N:]
    for j in range(2):
        sl = slice(j * _HALF, (j + 1) * _HALF)
        y = (jnp.dot(xl, w_ref[:_HALF, sl], preferred_element_type=jnp.float32)
             + jnp.dot(xr, w_ref[_HALF:, sl], preferred_element_type=jnp.float32)
             + b_ref[:, sl])
        m = jnp.mean(y, axis=0, keepdims=True)
        v = jnp.mean((y - m) ** 2, axis=0, keepdims=True)
        hn = (y - m) / jnp.sqrt(v + _EPS) * g_ref[:, sl] + be_ref[:, sl]
        o_ref[j * _N:(j + 1) * _N] = jnp.maximum(hn, 0.0)


_mlp_stage = pl.pallas_call(
    _mlp_stage_body,
    out_shape=jax.ShapeDtypeStruct((2 * _N, _HALF), jnp.float32),
)


def _pool_body(xst_ref, hst_ref, batch_ref,
               wp0_ref, bp0_ref, wp3_ref, bp3_ref, wo_ref, bo_ref, o_ref):
    gi = lax.broadcasted_iota(jnp.int32, (1, _G), 1)
    m = (batch_ref[...] == gi).astype(jnp.float32)
    dn = (((0,), (0,)), ((), ()))

    def pool_proj(st_ref, w_ref):
        pleft = lax.dot_general(m, st_ref[:_N], dn,
                                preferred_element_type=jnp.float32)
        pright = lax.dot_general(m, st_ref[_N:], dn,
                                 preferred_element_type=jnp.float32)
        return (jnp.dot(pleft, w_ref[:_HALF], preferred_element_type=jnp.float32)
                + jnp.dot(pright, w_ref[_HALF:], preferred_element_type=jnp.float32))

    oh = (pool_proj(xst_ref, wp0_ref) + pool_proj(hst_ref, wp3_ref)
          + bp0_ref[...] + bp3_ref[...])
    oh = jnp.maximum(oh, 0.0)
    o_ref[...] = jnp.dot(oh, wo_ref[...],
                         preferred_element_type=jnp.float32) + bo_ref[...]


_pool = pl.pallas_call(
    _pool_body,
    out_shape=jax.ShapeDtypeStruct((_G, _OUT), jnp.float32),
)


def kernel(x, edge_index, batch, atten_edge_index,
           l0_W1, l0_b1, l0_g1, l0_be1, l0_W2, l0_b2, l0_g2, l0_be2,
           l1_W1, l1_b1, l1_g1, l1_be1, l1_W2, l1_b2, l1_g2, l1_be2,
           l2_W1, l2_b1, l2_g1, l2_be1, l2_W2, l2_b2, l2_g2, l2_be2,
           Wp0, bp0, Wp3, bp3, Wo, bo):
    del atten_edge_index
    row = edge_index[0]
    col = jnp.concatenate([edge_index[1], edge_index[1] + _N])
    x_st = jnp.concatenate([x[:, :_HALF], x[:, _HALF:]], axis=0)
    batch2 = batch.reshape(_N, 1)

    layers = [
        (l0_W1, l0_b1, l0_g1, l0_be1, l0_W2, l0_b2, l0_g2, l0_be2),
        (l1_W1, l1_b1, l1_g1, l1_be1, l1_W2, l1_b2, l1_g2, l1_be2),
        (l2_W1, l2_b1, l2_g1, l2_be1, l2_W2, l2_b2, l2_g2, l2_be2),
    ]

    def r1(v):
        return v.reshape(1, -1)

    sc_segsum = _get_sc_segsum()
    h_st = x_st
    for (W1, b1, g1, be1, W2, b2, g2, be2) in layers:
        a_st = sc_segsum(h_st, col, row)
        t_st = _mlp_stage(a_st, W1, r1(b1), r1(g1), r1(be1))
        h_st = _mlp_stage(t_st, W2, r1(b2), r1(g2), r1(be2))

    return _pool(x_st, h_st, batch2,
                 Wp0, r1(bp0), Wp3, r1(bp3), Wo, r1(bo))


# speedup vs baseline: 2.2814x; 1.0602x over previous
import functools

import jax
import jax.numpy as jnp
from jax import lax
from jax.experimental import pallas as pl
from jax.experimental.pallas import tpu as pltpu
from jax.experimental.pallas import tpu_sc as plsc

_N = 10000
_E = 160000
_H = 256
_G = 128
_MID = 32
_OUT = 64
_HALF = 128
_NSUB = 16
_ECHUNKS = _E // 256
_ROWS_PER_SUB = 624
_TAIL = _N - _NSUB * _ROWS_PER_SUB
_CHUNKS_PER_SUB = -(-_ECHUNKS // _NSUB)
_EPS = 1e-5


def _sc_segsum_body(h_hbm, col_hbm, row_hbm, out_hbm,
                    acc_sh, colidx_v, rowidx_v, rows_v):
    c = lax.axis_index("c")
    s = lax.axis_index("s")

    pltpu.sync_copy(h_hbm.at[pl.ds(c * _N + s * _ROWS_PER_SUB, _ROWS_PER_SUB)],
                    acc_sh.at[pl.ds(s * _ROWS_PER_SUB, _ROWS_PER_SUB)])

    @pl.when(s == _NSUB - 1)
    def _():
        pltpu.sync_copy(h_hbm.at[pl.ds(c * _N + _NSUB * _ROWS_PER_SUB, _TAIL)],
                        acc_sh.at[pl.ds(_NSUB * _ROWS_PER_SUB, _TAIL)])

    plsc.subcore_barrier()

    @pl.loop(0, _CHUNKS_PER_SUB)
    def _edge_chunk(k):
        r = s + _NSUB * k

        @pl.when(r < _ECHUNKS)
        def _():
            pltpu.sync_copy(col_hbm.at[pl.ds(c * _E + r * 256, 256)], colidx_v)
            pltpu.sync_copy(row_hbm.at[pl.ds(r * 256, 256)], rowidx_v)
            pltpu.sync_copy(h_hbm.at[colidx_v], rows_v)
            pltpu.sync_copy(rows_v, acc_sh.at[rowidx_v], add=True)

    plsc.subcore_barrier()
    pltpu.sync_copy(acc_sh.at[pl.ds(s * _ROWS_PER_SUB, _ROWS_PER_SUB)],
                    out_hbm.at[pl.ds(c * _N + s * _ROWS_PER_SUB, _ROWS_PER_SUB)])

    @pl.when(s == _NSUB - 1)
    def _():
        pltpu.sync_copy(acc_sh.at[pl.ds(_NSUB * _ROWS_PER_SUB, _TAIL)],
                        out_hbm.at[pl.ds(c * _N + _NSUB * _ROWS_PER_SUB, _TAIL)])


@functools.cache
def _get_sc_segsum():
    return functools.partial(
        pl.kernel,
        out_type=jax.ShapeDtypeStruct((2 * _N, _HALF), jnp.float32),
        mesh=plsc.VectorSubcoreMesh(core_axis_name="c", subcore_axis_name="s"),
        scratch_types=[
            pltpu.VMEM_SHARED((_N, _HALF), jnp.float32),
            pltpu.VMEM((256,), jnp.int32),
            pltpu.VMEM((256,), jnp.int32),
            pltpu.VMEM((256, _HALF), jnp.float32),
        ],
    )(_sc_segsum_body)


def _stage(xl, xr, w_ref, b_ref, g_ref, be_ref):
    out = []
    for j in range(2):
        sl = slice(j * _HALF, (j + 1) * _HALF)
        y = (jnp.dot(xl, w_ref[:_HALF, sl], preferred_element_type=jnp.float32)
             + jnp.dot(xr, w_ref[_HALF:, sl], preferred_element_type=jnp.float32)
             + b_ref[:, sl])
        m = jnp.mean(y, axis=0, keepdims=True)
        v = jnp.mean((y - m) ** 2, axis=0, keepdims=True)
        hn = (y - m) / jnp.sqrt(v + _EPS) * g_ref[:, sl] + be_ref[:, sl]
        out.append(jnp.maximum(hn, 0.0))
    return out


def _mlp_layer_body(x_ref, w1_ref, b1_ref, g1_ref, be1_ref,
                    w2_ref, b2_ref, g2_ref, be2_ref, o_ref):
    tl, tr = _stage(x_ref[:_N], x_ref[_N:], w1_ref, b1_ref, g1_ref, be1_ref)
    hl, hr = _stage(tl, tr, w2_ref, b2_ref, g2_ref, be2_ref)
    o_ref[:_N] = hl
    o_ref[_N:] = hr


_mlp_layer = pl.pallas_call(
    _mlp_layer_body,
    out_shape=jax.ShapeDtypeStruct((2 * _N, _HALF), jnp.float32),
)


def _last_layer_pool_body(x_ref, w1_ref, b1_ref, g1_ref, be1_ref,
                          w2_ref, b2_ref, g2_ref, be2_ref,
                          xst_ref, batch_ref,
                          wp0_ref, bp0_ref, wp3_ref, bp3_ref, wo_ref, bo_ref,
                          o_ref):
    tl, tr = _stage(x_ref[:_N], x_ref[_N:], w1_ref, b1_ref, g1_ref, be1_ref)
    hl, hr = _stage(tl, tr, w2_ref, b2_ref, g2_ref, be2_ref)

    gi = lax.broadcasted_iota(jnp.int32, (1, _G), 1)
    m = (batch_ref[...] == gi).astype(jnp.float32)
    dn = (((0,), (0,)), ((), ()))

    def pool_proj(left, right, w_ref):
        pleft = lax.dot_general(m, left, dn, preferred_element_type=jnp.float32)
        pright = lax.dot_general(m, right, dn,
                                 preferred_element_type=jnp.float32)
        return (jnp.dot(pleft, w_ref[:_HALF], preferred_element_type=jnp.float32)
                + jnp.dot(pright, w_ref[_HALF:], preferred_element_type=jnp.float32))

    oh = (pool_proj(xst_ref[:_N], xst_ref[_N:], wp0_ref)
          + pool_proj(hl, hr, wp3_ref)
          + bp0_ref[...] + bp3_ref[...])
    oh = jnp.maximum(oh, 0.0)
    o_ref[...] = jnp.dot(oh, wo_ref[...],
                         preferred_element_type=jnp.float32) + bo_ref[...]


_last_layer_pool = pl.pallas_call(
    _last_layer_pool_body,
    out_shape=jax.ShapeDtypeStruct((_G, _OUT), jnp.float32),
)


def kernel(x, edge_index, batch, atten_edge_index,
           l0_W1, l0_b1, l0_g1, l0_be1, l0_W2, l0_b2, l0_g2, l0_be2,
           l1_W1, l1_b1, l1_g1, l1_be1, l1_W2, l1_b2, l1_g2, l1_be2,
           l2_W1, l2_b1, l2_g1, l2_be1, l2_W2, l2_b2, l2_g2, l2_be2,
           Wp0, bp0, Wp3, bp3, Wo, bo):
    del atten_edge_index
    row = edge_index[0]
    col = jnp.concatenate([edge_index[1], edge_index[1] + _N])
    x_st = jnp.concatenate([x[:, :_HALF], x[:, _HALF:]], axis=0)
    batch2 = batch.reshape(_N, 1)

    layers = [
        (l0_W1, l0_b1, l0_g1, l0_be1, l0_W2, l0_b2, l0_g2, l0_be2),
        (l1_W1, l1_b1, l1_g1, l1_be1, l1_W2, l1_b2, l1_g2, l1_be2),
        (l2_W1, l2_b1, l2_g1, l2_be1, l2_W2, l2_b2, l2_g2, l2_be2),
    ]

    def r1(v):
        return v.reshape(1, -1)

    sc_segsum = _get_sc_segsum()
    h_st = x_st
    for (W1, b1, g1, be1, W2, b2, g2, be2) in layers[:2]:
        a_st = sc_segsum(h_st, col, row)
        h_st = _mlp_layer(a_st, W1, r1(b1), r1(g1), r1(be1),
                          W2, r1(b2), r1(g2), r1(be2))

    (W1, b1, g1, be1, W2, b2, g2, be2) = layers[2]
    a_st = sc_segsum(h_st, col, row)
    return _last_layer_pool(a_st, W1, r1(b1), r1(g1), r1(be1),
                            W2, r1(b2), r1(g2), r1(be2),
                            x_st, batch2,
                            Wp0, r1(bp0), Wp3, r1(bp3), Wo, r1(bo))


# speedup vs baseline: 2.2826x; 1.0005x over previous
import functools

import jax
import jax.numpy as jnp
from jax import lax
from jax.experimental import pallas as pl
from jax.experimental.pallas import tpu as pltpu
from jax.experimental.pallas import tpu_sc as plsc

_N = 10000
_E = 160000
_H = 256
_G = 128
_MID = 32
_OUT = 64
_HALF = 128
_NSUB = 16
_ECHUNKS = _E // 256
_ROWS_PER_SUB = 624
_TAIL = _N - _NSUB * _ROWS_PER_SUB
_CHUNKS_PER_SUB = -(-_ECHUNKS // _NSUB)
_EPS = 1e-5


def _sc_segsum_body(h_hbm, col_hbm, row_hbm, out_hbm,
                    acc_sh, colidx_v, rowidx_v, rows_v):
    c = lax.axis_index("c")
    s = lax.axis_index("s")

    pltpu.sync_copy(h_hbm.at[pl.ds(c * _N + s * _ROWS_PER_SUB, _ROWS_PER_SUB)],
                    acc_sh.at[pl.ds(s * _ROWS_PER_SUB, _ROWS_PER_SUB)])

    @pl.when(s == _NSUB - 1)
    def _():
        pltpu.sync_copy(h_hbm.at[pl.ds(c * _N + _NSUB * _ROWS_PER_SUB, _TAIL)],
                        acc_sh.at[pl.ds(_NSUB * _ROWS_PER_SUB, _TAIL)])

    plsc.subcore_barrier()

    @pl.loop(0, _CHUNKS_PER_SUB)
    def _edge_chunk(k):
        r = s + _NSUB * k

        @pl.when(r < _ECHUNKS)
        def _():
            pltpu.sync_copy(col_hbm.at[pl.ds(c * _E + r * 256, 256)], colidx_v)
            pltpu.sync_copy(row_hbm.at[pl.ds(r * 256, 256)], rowidx_v)
            pltpu.sync_copy(h_hbm.at[colidx_v], rows_v)
            pltpu.sync_copy(rows_v, acc_sh.at[rowidx_v], add=True)

    plsc.subcore_barrier()
    pltpu.sync_copy(acc_sh.at[pl.ds(s * _ROWS_PER_SUB, _ROWS_PER_SUB)],
                    out_hbm.at[pl.ds(c * _N + s * _ROWS_PER_SUB, _ROWS_PER_SUB)])

    @pl.when(s == _NSUB - 1)
    def _():
        pltpu.sync_copy(acc_sh.at[pl.ds(_NSUB * _ROWS_PER_SUB, _TAIL)],
                        out_hbm.at[pl.ds(c * _N + _NSUB * _ROWS_PER_SUB, _TAIL)])


@functools.cache
def _get_sc_segsum():
    return functools.partial(
        pl.kernel,
        out_type=jax.ShapeDtypeStruct((2 * _N, _HALF), jnp.float32),
        mesh=plsc.VectorSubcoreMesh(core_axis_name="c", subcore_axis_name="s"),
        scratch_types=[
            pltpu.VMEM_SHARED((_N, _HALF), jnp.float32),
            pltpu.VMEM((256,), jnp.int32),
            pltpu.VMEM((256,), jnp.int32),
            pltpu.VMEM((256, _HALF), jnp.float32),
        ],
    )(_sc_segsum_body)


def _stage(xl, xr, w_ref, b_ref, g_ref, be_ref):
    out = []
    for j in range(2):
        sl = slice(j * _HALF, (j + 1) * _HALF)
        y = (jnp.dot(xl, w_ref[:_HALF, sl], preferred_element_type=jnp.float32)
             + jnp.dot(xr, w_ref[_HALF:, sl], preferred_element_type=jnp.float32)
             + b_ref[:, sl])
        m = jnp.mean(y, axis=0, keepdims=True)
        v = jnp.mean((y - m) ** 2, axis=0, keepdims=True)
        hn = (y - m) / jnp.sqrt(v + _EPS) * g_ref[:, sl] + be_ref[:, sl]
        out.append(jnp.maximum(hn, 0.0))
    return out


def _mlp_layer_body(x_ref, w1_ref, b1_ref, g1_ref, be1_ref,
                    w2_ref, b2_ref, g2_ref, be2_ref, o_ref):
    tl, tr = _stage(x_ref[:_N], x_ref[_N:], w1_ref, b1_ref, g1_ref, be1_ref)
    hl, hr = _stage(tl, tr, w2_ref, b2_ref, g2_ref, be2_ref)
    o_ref[:_N] = hl
    o_ref[_N:] = hr


_mlp_layer = pl.pallas_call(
    _mlp_layer_body,
    out_shape=jax.ShapeDtypeStruct((2 * _N, _HALF), jnp.float32),
)


def _pool_mm(m, left, right, w_ref):
    dn = (((0,), (0,)), ((), ()))
    pleft = lax.dot_general(m, left, dn, preferred_element_type=jnp.float32)
    pright = lax.dot_general(m, right, dn, preferred_element_type=jnp.float32)
    return (jnp.dot(pleft, w_ref[:_HALF], preferred_element_type=jnp.float32)
            + jnp.dot(pright, w_ref[_HALF:], preferred_element_type=jnp.float32))


def _onehot(batch_ref):
    gi = lax.broadcasted_iota(jnp.int32, (1, _G), 1)
    return (batch_ref[...] == gi).astype(jnp.float32)


def _pool0_body(xst_ref, batch_ref, wp0_ref, bp0_ref, o_ref):
    m = _onehot(batch_ref)
    o_ref[...] = _pool_mm(m, xst_ref[:_N], xst_ref[_N:], wp0_ref) + bp0_ref[...]


_pool0 = pl.pallas_call(
    _pool0_body,
    out_shape=jax.ShapeDtypeStruct((_G, _MID), jnp.float32),
)


def _last_layer_pool_body(x_ref, w1_ref, b1_ref, g1_ref, be1_ref,
                          w2_ref, b2_ref, g2_ref, be2_ref,
                          p0_ref, batch_ref,
                          wp3_ref, bp3_ref, wo_ref, bo_ref,
                          o_ref):
    tl, tr = _stage(x_ref[:_N], x_ref[_N:], w1_ref, b1_ref, g1_ref, be1_ref)
    hl, hr = _stage(tl, tr, w2_ref, b2_ref, g2_ref, be2_ref)

    m = _onehot(batch_ref)
    oh = p0_ref[...] + _pool_mm(m, hl, hr, wp3_ref) + bp3_ref[...]
    oh = jnp.maximum(oh, 0.0)
    o_ref[...] = jnp.dot(oh, wo_ref[...],
                         preferred_element_type=jnp.float32) + bo_ref[...]


_last_layer_pool = pl.pallas_call(
    _last_layer_pool_body,
    out_shape=jax.ShapeDtypeStruct((_G, _OUT), jnp.float32),
)


def kernel(x, edge_index, batch, atten_edge_index,
           l0_W1, l0_b1, l0_g1, l0_be1, l0_W2, l0_b2, l0_g2, l0_be2,
           l1_W1, l1_b1, l1_g1, l1_be1, l1_W2, l1_b2, l1_g2, l1_be2,
           l2_W1, l2_b1, l2_g1, l2_be1, l2_W2, l2_b2, l2_g2, l2_be2,
           Wp0, bp0, Wp3, bp3, Wo, bo):
    del atten_edge_index
    row = edge_index[0]
    col = jnp.concatenate([edge_index[1], edge_index[1] + _N])
    x_st = jnp.concatenate([x[:, :_HALF], x[:, _HALF:]], axis=0)
    batch2 = batch.reshape(_N, 1)

    layers = [
        (l0_W1, l0_b1, l0_g1, l0_be1, l0_W2, l0_b2, l0_g2, l0_be2),
        (l1_W1, l1_b1, l1_g1, l1_be1, l1_W2, l1_b2, l1_g2, l1_be2),
        (l2_W1, l2_b1, l2_g1, l2_be1, l2_W2, l2_b2, l2_g2, l2_be2),
    ]

    def r1(v):
        return v.reshape(1, -1)

    sc_segsum = _get_sc_segsum()
    p0 = _pool0(x_st, batch2, Wp0, r1(bp0))
    h_st = x_st
    for (W1, b1, g1, be1, W2, b2, g2, be2) in layers[:2]:
        a_st = sc_segsum(h_st, col, row)
        h_st = _mlp_layer(a_st, W1, r1(b1), r1(g1), r1(be1),
                          W2, r1(b2), r1(g2), r1(be2))

    (W1, b1, g1, be1, W2, b2, g2, be2) = layers[2]
    a_st = sc_segsum(h_st, col, row)
    return _last_layer_pool(a_st, W1, r1(b1), r1(g1), r1(be1),
                            W2, r1(b2), r1(g2), r1(be2),
                            p0, batch2,
                            Wp3, r1(bp3), Wo, r1(bo))
